# Initial kernel scaffold; baseline (speedup 1.0000x reference)
import functools

import jax
import jax.numpy as jnp
from jax import lax
from jax.experimental import pallas as pl
from jax.experimental.pallas import tpu as pltpu
from jax.experimental.pallas import tpu_sc as plsc

N = 10000
E = 320000
H = 128
G = 64
T = 8

NC = 2
NS = 16
NW = NC * NS
NPAD = 10240
EPAD = 327680
EPW = EPAD // NW
CH = 128
NCHUNK = EPW // CH
RPT = NPAD // NS
DCH = 128
DEGW = 16

_mesh = plsc.VectorSubcoreMesh(
    core_axis_name="c", subcore_axis_name="s", num_cores=NC, num_subcores=NS)


@functools.partial(
    pl.kernel,
    out_type=jax.ShapeDtypeStruct((NC, NPAD, DEGW), jnp.float32),
    mesh=_mesh,
    scratch_types=[
        pltpu.VMEM((CH,), jnp.int32),
        pltpu.VMEM((CH, DEGW), jnp.float32),
        pltpu.VMEM((RPT, DEGW), jnp.float32),
    ],
)
def _sc_deg(dst_hbm, deg_hbm, didx, ones_v, buf):
    c = lax.axis_index("c")
    s = lax.axis_index("s")
    wid = c * NS + s

    def fill_ones(i, _):
        ones_v[i] = jnp.ones((DEGW,), jnp.float32)
        return 0
    lax.fori_loop(0, CH, fill_ones, 0)

    def fill_zero(i, _):
        buf[i] = jnp.zeros((DEGW,), jnp.float32)
        return 0
    lax.fori_loop(0, RPT, fill_zero, 0)

    pltpu.sync_copy(buf, deg_hbm.at[c, pl.ds(s * RPT, RPT)])
    plsc.subcore_barrier()

    def body(i, _):
        base = wid * EPW + i * CH
        pltpu.sync_copy(dst_hbm.at[pl.ds(base, CH)], didx)
        pltpu.sync_copy(ones_v, deg_hbm.at[c].at[didx], add=True)
        return 0
    lax.fori_loop(0, NCHUNK, body, 0)


@functools.partial(
    pl.kernel,
    out_type=jax.ShapeDtypeStruct((NC, NPAD, H), jnp.float32),
    mesh=_mesh,
    scratch_types=[
        pltpu.VMEM((CH,), jnp.int32),
        pltpu.VMEM((CH,), jnp.int32),
        pltpu.VMEM((CH, H), jnp.float32),
        pltpu.VMEM_SHARED((NPAD, H), jnp.float32),
        pltpu.SemaphoreType.DMA,
    ],
)
def _sc_agg(g_hbm, src_hbm, dst_hbm, p_hbm, sidx, didx, rows, acc, sem):
    c = lax.axis_index("c")
    s = lax.axis_index("s")
    wid = c * NS + s

    def fill_zero(i, _):
        rows[i // 8, pl.ds((i % 8) * 16, 16)] = jnp.zeros((16,), jnp.float32)
        return 0
    lax.fori_loop(0, CH * (H // 16), fill_zero, 0)

    def zero_acc(j, _):
        pltpu.sync_copy(rows, acc.at[pl.ds(s * RPT + j * DCH, DCH)])
        return 0
    lax.fori_loop(0, RPT // DCH, zero_acc, 0)
    plsc.subcore_barrier()

    def body(i, _):
        base = wid * EPW + i * CH
        pltpu.sync_copy(src_hbm.at[pl.ds(base, CH)], sidx)
        pltpu.async_copy(g_hbm.at[sidx], rows, sem).wait()
        pltpu.sync_copy(dst_hbm.at[pl.ds(base, CH)], didx)
        pltpu.sync_copy(rows, acc.at[didx], add=True)
        return 0
    lax.fori_loop(0, NCHUNK, body, 0)

    plsc.subcore_barrier()

    def dump(j, _):
        r0 = s * RPT + j * DCH
        pltpu.sync_copy(acc.at[pl.ds(r0, DCH)], rows)
        pltpu.sync_copy(rows, p_hbm.at[c, pl.ds(r0, DCH)])
        return 0
    lax.fori_loop(0, RPT // DCH, dump, 0)


BR = 1024
GRID = NPAD // BR


def _tc1_body(x_ref, w_ref, d0_ref, d1_ref, g_ref, dis_ref):
    deg = d0_ref[...] + d1_ref[...] + 1.0
    dis = lax.rsqrt(deg)
    t = jnp.dot(x_ref[...], w_ref[...], preferred_element_type=jnp.float32)
    g_ref[...] = t * dis
    dis_ref[...] = dis


_tc1 = pl.pallas_call(
    _tc1_body,
    grid=(GRID,),
    in_specs=[
        pl.BlockSpec((BR, H), lambda i: (i, 0)),
        pl.BlockSpec((H, H), lambda i: (0, 0)),
        pl.BlockSpec((BR, 1), lambda i: (i, 0)),
        pl.BlockSpec((BR, 1), lambda i: (i, 0)),
    ],
    out_specs=[
        pl.BlockSpec((BR, H), lambda i: (i, 0)),
        pl.BlockSpec((BR, 1), lambda i: (i, 0)),
    ],
    out_shape=[
        jax.ShapeDtypeStruct((NPAD, H), jnp.float32),
        jax.ShapeDtypeStruct((NPAD, 1), jnp.float32),
    ],
)


def _tc_mid_body(p0_ref, p1_ref, g_ref, dis_ref, w_ref, b_ref, out_ref):
    d = dis_ref[...]
    h = jnp.maximum(d * (p0_ref[...] + p1_ref[...] + g_ref[...]) + b_ref[...], 0.0)
    out_ref[...] = d * jnp.dot(h, w_ref[...], preferred_element_type=jnp.float32)


_tc_mid = pl.pallas_call(
    _tc_mid_body,
    grid=(GRID,),
    in_specs=[
        pl.BlockSpec((BR, H), lambda i: (i, 0)),
        pl.BlockSpec((BR, H), lambda i: (i, 0)),
        pl.BlockSpec((BR, H), lambda i: (i, 0)),
        pl.BlockSpec((BR, 1), lambda i: (i, 0)),
        pl.BlockSpec((H, H), lambda i: (0, 0)),
        pl.BlockSpec((1, H), lambda i: (0, 0)),
    ],
    out_specs=pl.BlockSpec((BR, H), lambda i: (i, 0)),
    out_shape=jax.ShapeDtypeStruct((NPAD, H), jnp.float32),
)


def _tc_final_body(p0_ref, p1_ref, g_ref, dis_ref, b_ref, batch_ref, hw_ref,
                   hb_ref, out_ref, sums, cnts):
    i = pl.program_id(0)

    @pl.when(i == 0)
    def _():
        sums[...] = jnp.zeros_like(sums)
        cnts[...] = jnp.zeros_like(cnts)

    d = dis_ref[...]
    h3 = d * (p0_ref[...] + p1_ref[...] + g_ref[...]) + b_ref[...]
    onehot = (batch_ref[...] == lax.broadcasted_iota(jnp.int32, (BR, G), 1)
              ).astype(jnp.float32)
    sums[...] += lax.dot_general(onehot, h3, (((0,), (0,)), ((), ())),
                                 preferred_element_type=jnp.float32)
    cnts[...] += lax.dot_general(onehot, jnp.ones((BR, H), jnp.float32),
                                 (((0,), (0,)), ((), ())),
                                 preferred_element_type=jnp.float32)

    @pl.when(i == GRID - 1)
    def _():
        pooled = sums[...] / jnp.maximum(cnts[...], 1.0)
        out_ref[...] = jnp.dot(pooled, hw_ref[...],
                               preferred_element_type=jnp.float32) + hb_ref[...]


_tc_final = pl.pallas_call(
    _tc_final_body,
    grid=(GRID,),
    in_specs=[
        pl.BlockSpec((BR, H), lambda i: (i, 0)),
        pl.BlockSpec((BR, H), lambda i: (i, 0)),
        pl.BlockSpec((BR, H), lambda i: (i, 0)),
        pl.BlockSpec((BR, 1), lambda i: (i, 0)),
        pl.BlockSpec((1, H), lambda i: (0, 0)),
        pl.BlockSpec((BR, 1), lambda i: (i, 0)),
        pl.BlockSpec((H, T), lambda i: (0, 0)),
        pl.BlockSpec((1, T), lambda i: (0, 0)),
    ],
    out_specs=pl.BlockSpec((G, T), lambda i: (0, 0)),
    out_shape=jax.ShapeDtypeStruct((G, T), jnp.float32),
    scratch_shapes=[
        pltpu.VMEM((G, H), jnp.float32),
        pltpu.VMEM((G, H), jnp.float32),
    ],
)


def kernel(x, edge_index, batch, r_target, W1, b1, W2, b2, W3, b3, head_W, head_b):
    del r_target
    pad_idx = jnp.full((EPAD - E,), N, dtype=jnp.int32)
    srcp = jnp.concatenate([edge_index[0].astype(jnp.int32), pad_idx])
    dstp = jnp.concatenate([edge_index[1].astype(jnp.int32), pad_idx])
    xp = jnp.zeros((NPAD, H), jnp.float32).at[:N].set(x)
    batchp = jnp.concatenate(
        [batch.astype(jnp.int32), jnp.full((NPAD - N,), G, jnp.int32)]
    ).reshape(NPAD, 1)

    degh = _sc_deg(dstp)
    deg0 = degh[0, :, 0:1]
    deg1 = degh[1, :, 0:1]

    g1, dis = _tc1(xp, W1, deg0, deg1)
    p = _sc_agg(g1, srcp, dstp)
    g2 = _tc_mid(p[0], p[1], g1, dis, W2, b1.reshape(1, H))
    p = _sc_agg(g2, srcp, dstp)
    g3 = _tc_mid(p[0], p[1], g2, dis, W3, b2.reshape(1, H))
    p = _sc_agg(g3, srcp, dstp)
    out = _tc_final(p[0], p[1], g3, dis, b3.reshape(1, H), batchp,
                    head_W, head_b.reshape(1, T))
    return out


# speedup vs baseline: 5.9800x; 5.9800x over previous
import functools

import jax
import jax.numpy as jnp
from jax import lax
from jax.experimental import pallas as pl
from jax.experimental.pallas import tpu as pltpu
from jax.experimental.pallas import tpu_sc as plsc

N = 10000
E = 320000
H = 128
G = 64
T = 8

NC = 2
NS = 16
NW = NC * NS
NPAD = 10240
EPAD = 327680
EPW = EPAD // NW
CH = 128
NCHUNK = EPW // CH
RPT = NPAD // NS
DCH = 128

_mesh = plsc.VectorSubcoreMesh(
    core_axis_name="c", subcore_axis_name="s", num_cores=NC, num_subcores=NS)


@functools.partial(
    pl.kernel,
    out_type=jax.ShapeDtypeStruct((NC, NPAD, H), jnp.float32),
    mesh=_mesh,
    scratch_types=[
        pltpu.VMEM((CH,), jnp.int32),
        pltpu.VMEM((CH, H), jnp.float32),
        pltpu.VMEM_SHARED((NPAD, H), jnp.float32),
    ],
)
def _sc_deg(dst_hbm, deg_hbm, didx, ones_v, acc):
    c = lax.axis_index("c")
    s = lax.axis_index("s")
    wid = c * NS + s

    def fill_zero(i, _):
        ones_v[i // 8, pl.ds((i % 8) * 16, 16)] = jnp.zeros((16,), jnp.float32)
        return 0
    lax.fori_loop(0, CH * (H // 16), fill_zero, 0)

    def zero_acc(j, _):
        pltpu.sync_copy(ones_v, acc.at[pl.ds(s * RPT + j * DCH, DCH)])
        return 0
    lax.fori_loop(0, RPT // DCH, zero_acc, 0)

    def fill_ones(i, _):
        ones_v[i // 8, pl.ds((i % 8) * 16, 16)] = jnp.ones((16,), jnp.float32)
        return 0
    lax.fori_loop(0, CH * (H // 16), fill_ones, 0)
    plsc.subcore_barrier()

    def body(i, _):
        base = wid * EPW + i * CH
        pltpu.sync_copy(dst_hbm.at[pl.ds(base, CH)], didx)
        pltpu.sync_copy(ones_v, acc.at[didx], add=True)
        return 0
    lax.fori_loop(0, NCHUNK, body, 0)

    plsc.subcore_barrier()

    def dump(j, _):
        r0 = s * RPT + j * DCH
        pltpu.sync_copy(acc.at[pl.ds(r0, DCH)], ones_v)
        pltpu.sync_copy(ones_v, deg_hbm.at[c, pl.ds(r0, DCH)])
        return 0
    lax.fori_loop(0, RPT // DCH, dump, 0)


@functools.partial(
    pl.kernel,
    out_type=jax.ShapeDtypeStruct((NC, NPAD, H), jnp.float32),
    mesh=_mesh,
    scratch_types=[
        pltpu.VMEM((CH,), jnp.int32),
        pltpu.VMEM((CH,), jnp.int32),
        pltpu.VMEM((CH, H), jnp.float32),
        pltpu.VMEM_SHARED((NPAD, H), jnp.float32),
        pltpu.SemaphoreType.DMA,
    ],
)
def _sc_agg(g_hbm, src_hbm, dst_hbm, p_hbm, sidx, didx, rows, acc, sem):
    c = lax.axis_index("c")
    s = lax.axis_index("s")
    wid = c * NS + s

    def fill_zero(i, _):
        rows[i // 8, pl.ds((i % 8) * 16, 16)] = jnp.zeros((16,), jnp.float32)
        return 0
    lax.fori_loop(0, CH * (H // 16), fill_zero, 0)

    def zero_acc(j, _):
        pltpu.sync_copy(rows, acc.at[pl.ds(s * RPT + j * DCH, DCH)])
        return 0
    lax.fori_loop(0, RPT // DCH, zero_acc, 0)
    plsc.subcore_barrier()

    def body(i, _):
        base = wid * EPW + i * CH
        pltpu.sync_copy(src_hbm.at[pl.ds(base, CH)], sidx)
        pltpu.async_copy(g_hbm.at[sidx], rows, sem).wait()
        pltpu.sync_copy(dst_hbm.at[pl.ds(base, CH)], didx)
        pltpu.sync_copy(rows, acc.at[didx], add=True)
        return 0
    lax.fori_loop(0, NCHUNK, body, 0)

    plsc.subcore_barrier()

    def dump(j, _):
        r0 = s * RPT + j * DCH
        pltpu.sync_copy(acc.at[pl.ds(r0, DCH)], rows)
        pltpu.sync_copy(rows, p_hbm.at[c, pl.ds(r0, DCH)])
        return 0
    lax.fori_loop(0, RPT // DCH, dump, 0)


BR = 1024
GRID = NPAD // BR


def _tc1_body(x_ref, w_ref, d0_ref, d1_ref, g_ref, dis_ref):
    deg = d0_ref[...] + d1_ref[...] + 1.0
    dis = lax.rsqrt(deg)
    t = jnp.dot(x_ref[...], w_ref[...], preferred_element_type=jnp.float32)
    g_ref[...] = t * dis
    dis_ref[...] = dis


_tc1 = pl.pallas_call(
    _tc1_body,
    grid=(GRID,),
    in_specs=[
        pl.BlockSpec((BR, H), lambda i: (i, 0)),
        pl.BlockSpec((H, H), lambda i: (0, 0)),
        pl.BlockSpec((BR, 1), lambda i: (i, 0)),
        pl.BlockSpec((BR, 1), lambda i: (i, 0)),
    ],
    out_specs=[
        pl.BlockSpec((BR, H), lambda i: (i, 0)),
        pl.BlockSpec((BR, 1), lambda i: (i, 0)),
    ],
    out_shape=[
        jax.ShapeDtypeStruct((NPAD, H), jnp.float32),
        jax.ShapeDtypeStruct((NPAD, 1), jnp.float32),
    ],
)


def _tc_mid_body(p0_ref, p1_ref, g_ref, dis_ref, w_ref, b_ref, out_ref):
    d = dis_ref[...]
    h = jnp.maximum(d * (p0_ref[...] + p1_ref[...] + g_ref[...]) + b_ref[...], 0.0)
    out_ref[...] = d * jnp.dot(h, w_ref[...], preferred_element_type=jnp.float32)


_tc_mid = pl.pallas_call(
    _tc_mid_body,
    grid=(GRID,),
    in_specs=[
        pl.BlockSpec((BR, H), lambda i: (i, 0)),
        pl.BlockSpec((BR, H), lambda i: (i, 0)),
        pl.BlockSpec((BR, H), lambda i: (i, 0)),
        pl.BlockSpec((BR, 1), lambda i: (i, 0)),
        pl.BlockSpec((H, H), lambda i: (0, 0)),
        pl.BlockSpec((1, H), lambda i: (0, 0)),
    ],
    out_specs=pl.BlockSpec((BR, H), lambda i: (i, 0)),
    out_shape=jax.ShapeDtypeStruct((NPAD, H), jnp.float32),
)


def _tc_final_body(p0_ref, p1_ref, g_ref, dis_ref, b_ref, batch_ref, hw_ref,
                   hb_ref, out_ref, sums, cnts):
    i = pl.program_id(0)

    @pl.when(i == 0)
    def _():
        sums[...] = jnp.zeros_like(sums)
        cnts[...] = jnp.zeros_like(cnts)

    d = dis_ref[...]
    h3 = d * (p0_ref[...] + p1_ref[...] + g_ref[...]) + b_ref[...]
    onehot = (batch_ref[...] == lax.broadcasted_iota(jnp.int32, (BR, G), 1)
              ).astype(jnp.float32)
    sums[...] += lax.dot_general(onehot, h3, (((0,), (0,)), ((), ())),
                                 preferred_element_type=jnp.float32)
    cnts[...] += lax.dot_general(onehot, jnp.ones((BR, H), jnp.float32),
                                 (((0,), (0,)), ((), ())),
                                 preferred_element_type=jnp.float32)

    @pl.when(i == GRID - 1)
    def _():
        pooled = sums[...] / jnp.maximum(cnts[...], 1.0)
        out_ref[...] = jnp.dot(pooled, hw_ref[...],
                               preferred_element_type=jnp.float32) + hb_ref[...]


_tc_final = pl.pallas_call(
    _tc_final_body,
    grid=(GRID,),
    in_specs=[
        pl.BlockSpec((BR, H), lambda i: (i, 0)),
        pl.BlockSpec((BR, H), lambda i: (i, 0)),
        pl.BlockSpec((BR, H), lambda i: (i, 0)),
        pl.BlockSpec((BR, 1), lambda i: (i, 0)),
        pl.BlockSpec((1, H), lambda i: (0, 0)),
        pl.BlockSpec((BR, 1), lambda i: (i, 0)),
        pl.BlockSpec((H, T), lambda i: (0, 0)),
        pl.BlockSpec((1, T), lambda i: (0, 0)),
    ],
    out_specs=pl.BlockSpec((G, T), lambda i: (0, 0)),
    out_shape=jax.ShapeDtypeStruct((G, T), jnp.float32),
    scratch_shapes=[
        pltpu.VMEM((G, H), jnp.float32),
        pltpu.VMEM((G, H), jnp.float32),
    ],
)


def kernel(x, edge_index, batch, r_target, W1, b1, W2, b2, W3, b3, head_W, head_b):
    del r_target
    pad_idx = jnp.full((EPAD - E,), N, dtype=jnp.int32)
    srcp = jnp.concatenate([edge_index[0].astype(jnp.int32), pad_idx])
    dstp = jnp.concatenate([edge_index[1].astype(jnp.int32), pad_idx])
    xp = jnp.zeros((NPAD, H), jnp.float32).at[:N].set(x)
    batchp = jnp.concatenate(
        [batch.astype(jnp.int32), jnp.full((NPAD - N,), G, jnp.int32)]
    ).reshape(NPAD, 1)

    degh = _sc_deg(dstp)
    deg0 = degh[0, :, 0:1]
    deg1 = degh[1, :, 0:1]

    g1, dis = _tc1(xp, W1, deg0, deg1)
    p = _sc_agg(g1, srcp, dstp)
    g2 = _tc_mid(p[0], p[1], g1, dis, W2, b1.reshape(1, H))
    p = _sc_agg(g2, srcp, dstp)
    g3 = _tc_mid(p[0], p[1], g2, dis, W3, b2.reshape(1, H))
    p = _sc_agg(g3, srcp, dstp)
    out = _tc_final(p[0], p[1], g3, dis, b3.reshape(1, H), batchp,
                    head_W, head_b.reshape(1, T))
    return out


# speedup vs baseline: 12.6375x; 2.1133x over previous
import functools

import jax
import jax.numpy as jnp
from jax import lax
from jax.experimental import pallas as pl
from jax.experimental.pallas import tpu as pltpu
from jax.experimental.pallas import tpu_sc as plsc

N = 10000
E = 320000
H = 128
G = 64
T = 8

NC = 2
NS = 16
NW = NC * NS
NPAD = 10240
EPAD = 327680
EPW = EPAD // NW
CH = 128
NCHUNK = EPW // CH
RPT = NPAD // NS
DCH = 128

_mesh = plsc.VectorSubcoreMesh(
    core_axis_name="c", subcore_axis_name="s", num_cores=NC, num_subcores=NS)


@functools.partial(
    pl.kernel,
    out_type=jax.ShapeDtypeStruct((NC, NPAD, H), jnp.float32),
    mesh=_mesh,
    scratch_types=[
        pltpu.VMEM((CH,), jnp.int32),
        pltpu.VMEM((CH, H), jnp.float32),
        pltpu.VMEM_SHARED((NPAD, H), jnp.float32),
    ],
)
def _sc_deg(dst_hbm, deg_hbm, didx, ones_v, acc):
    c = lax.axis_index("c")
    s = lax.axis_index("s")
    wid = c * NS + s

    def fill_zero(i, _):
        ones_v[i // 8, pl.ds((i % 8) * 16, 16)] = jnp.zeros((16,), jnp.float32)
        return 0
    lax.fori_loop(0, CH * (H // 16), fill_zero, 0)

    def zero_acc(j, _):
        pltpu.sync_copy(ones_v, acc.at[pl.ds(s * RPT + j * DCH, DCH)])
        return 0
    lax.fori_loop(0, RPT // DCH, zero_acc, 0)

    def fill_ones(i, _):
        ones_v[i // 8, pl.ds((i % 8) * 16, 16)] = jnp.ones((16,), jnp.float32)
        return 0
    lax.fori_loop(0, CH * (H // 16), fill_ones, 0)
    plsc.subcore_barrier()

    def body(i, _):
        base = wid * EPW + i * CH
        pltpu.sync_copy(dst_hbm.at[pl.ds(base, CH)], didx)
        pltpu.sync_copy(ones_v, acc.at[didx], add=True)
        return 0
    lax.fori_loop(0, NCHUNK, body, 0)

    plsc.subcore_barrier()

    def dump(j, _):
        r0 = s * RPT + j * DCH
        pltpu.sync_copy(acc.at[pl.ds(r0, DCH)], ones_v)
        pltpu.sync_copy(ones_v, deg_hbm.at[c, pl.ds(r0, DCH)])
        return 0
    lax.fori_loop(0, RPT // DCH, dump, 0)


@functools.partial(
    pl.kernel,
    out_type=jax.ShapeDtypeStruct((NC, NPAD, H), jnp.float32),
    mesh=_mesh,
    scratch_types=[
        pltpu.VMEM((CH,), jnp.int32),
        pltpu.VMEM((CH,), jnp.int32),
        pltpu.VMEM((CH, H), jnp.float32),
        pltpu.VMEM_SHARED((NPAD, H), jnp.float32),
        pltpu.SemaphoreType.DMA,
    ],
)
def _sc_agg(g_hbm, src_hbm, dst_hbm, p_hbm, sidx, didx, rows, acc, sem):
    c = lax.axis_index("c")
    s = lax.axis_index("s")
    wid = c * NS + s

    def fill_zero(i, _):
        rows[i // 8, pl.ds((i % 8) * 16, 16)] = jnp.zeros((16,), jnp.float32)
        return 0
    lax.fori_loop(0, CH * (H // 16), fill_zero, 0)

    def zero_acc(j, _):
        pltpu.sync_copy(rows, acc.at[pl.ds(s * RPT + j * DCH, DCH)])
        return 0
    lax.fori_loop(0, RPT // DCH, zero_acc, 0)
    plsc.subcore_barrier()

    def body(i, _):
        base = wid * EPW + i * CH
        pltpu.sync_copy(src_hbm.at[pl.ds(base, CH)], sidx)
        pltpu.async_copy(g_hbm.at[sidx], rows, sem).wait()
        pltpu.sync_copy(dst_hbm.at[pl.ds(base, CH)], didx)
        pltpu.sync_copy(rows, acc.at[didx], add=True)
        return 0
    lax.fori_loop(0, NCHUNK, body, 0)

    plsc.subcore_barrier()

    def dump(j, _):
        r0 = s * RPT + j * DCH
        pltpu.sync_copy(acc.at[pl.ds(r0, DCH)], rows)
        pltpu.sync_copy(rows, p_hbm.at[c, pl.ds(r0, DCH)])
        return 0
    lax.fori_loop(0, RPT // DCH, dump, 0)


BR = 1024
GRID = NPAD // BR


def _tc1_body(x_ref, w_ref, d0_ref, d1_ref, g_ref, dis_ref):
    deg = d0_ref[...] + d1_ref[...] + 1.0
    dis = lax.rsqrt(deg)
    t = jnp.dot(x_ref[...], w_ref[...], preferred_element_type=jnp.float32)
    g_ref[...] = t * dis
    dis_ref[...] = dis


_tc1 = pl.pallas_call(
    _tc1_body,
    grid=(GRID,),
    in_specs=[
        pl.BlockSpec((BR, H), lambda i: (i, 0)),
        pl.BlockSpec((H, H), lambda i: (0, 0)),
        pl.BlockSpec((BR, 1), lambda i: (i, 0)),
        pl.BlockSpec((BR, 1), lambda i: (i, 0)),
    ],
    out_specs=[
        pl.BlockSpec((BR, H), lambda i: (i, 0)),
        pl.BlockSpec((BR, 1), lambda i: (i, 0)),
    ],
    out_shape=[
        jax.ShapeDtypeStruct((NPAD, H), jnp.float32),
        jax.ShapeDtypeStruct((NPAD, 1), jnp.float32),
    ],
)


def _tc_mid_body(p0_ref, p1_ref, g_ref, dis_ref, w_ref, b_ref, out_ref):
    d = dis_ref[...]
    h = jnp.maximum(d * (p0_ref[...] + p1_ref[...] + g_ref[...]) + b_ref[...], 0.0)
    out_ref[...] = d * jnp.dot(h, w_ref[...], preferred_element_type=jnp.float32)


_tc_mid = pl.pallas_call(
    _tc_mid_body,
    grid=(GRID,),
    in_specs=[
        pl.BlockSpec((BR, H), lambda i: (i, 0)),
        pl.BlockSpec((BR, H), lambda i: (i, 0)),
        pl.BlockSpec((BR, H), lambda i: (i, 0)),
        pl.BlockSpec((BR, 1), lambda i: (i, 0)),
        pl.BlockSpec((H, H), lambda i: (0, 0)),
        pl.BlockSpec((1, H), lambda i: (0, 0)),
    ],
    out_specs=pl.BlockSpec((BR, H), lambda i: (i, 0)),
    out_shape=jax.ShapeDtypeStruct((NPAD, H), jnp.float32),
)


def _tc_final_body(p0_ref, p1_ref, g_ref, dis_ref, b_ref, batch_ref, hw_ref,
                   hb_ref, out_ref, sums, cnts):
    i = pl.program_id(0)

    @pl.when(i == 0)
    def _():
        sums[...] = jnp.zeros_like(sums)
        cnts[...] = jnp.zeros_like(cnts)

    d = dis_ref[...]
    h3 = d * (p0_ref[...] + p1_ref[...] + g_ref[...]) + b_ref[...]
    onehot = (batch_ref[...] == lax.broadcasted_iota(jnp.int32, (BR, G), 1)
              ).astype(jnp.float32)
    sums[...] += lax.dot_general(onehot, h3, (((0,), (0,)), ((), ())),
                                 preferred_element_type=jnp.float32)
    cnts[...] += lax.dot_general(onehot, jnp.ones((BR, H), jnp.float32),
                                 (((0,), (0,)), ((), ())),
                                 preferred_element_type=jnp.float32)

    @pl.when(i == GRID - 1)
    def _():
        pooled = sums[...] / jnp.maximum(cnts[...], 1.0)
        out_ref[...] = jnp.dot(pooled, hw_ref[...],
                               preferred_element_type=jnp.float32) + hb_ref[...]


_tc_final = pl.pallas_call(
    _tc_final_body,
    grid=(GRID,),
    in_specs=[
        pl.BlockSpec((BR, H), lambda i: (i, 0)),
        pl.BlockSpec((BR, H), lambda i: (i, 0)),
        pl.BlockSpec((BR, H), lambda i: (i, 0)),
        pl.BlockSpec((BR, 1), lambda i: (i, 0)),
        pl.BlockSpec((1, H), lambda i: (0, 0)),
        pl.BlockSpec((BR, 1), lambda i: (i, 0)),
        pl.BlockSpec((H, T), lambda i: (0, 0)),
        pl.BlockSpec((1, T), lambda i: (0, 0)),
    ],
    out_specs=pl.BlockSpec((G, T), lambda i: (0, 0)),
    out_shape=jax.ShapeDtypeStruct((G, T), jnp.float32),
    scratch_shapes=[
        pltpu.VMEM((G, H), jnp.float32),
        pltpu.VMEM((G, H), jnp.float32),
    ],
)


def kernel(x, edge_index, batch, r_target, W1, b1, W2, b2, W3, b3, head_W, head_b):
    del r_target
    pad_idx = N + jnp.arange(EPAD - E, dtype=jnp.int32) % (NPAD - N)
    srcp = jnp.concatenate([edge_index[0].astype(jnp.int32), pad_idx])
    dstp = jnp.concatenate([edge_index[1].astype(jnp.int32), pad_idx])
    xp = jnp.zeros((NPAD, H), jnp.float32).at[:N].set(x)
    batchp = jnp.concatenate(
        [batch.astype(jnp.int32), jnp.full((NPAD - N,), G, jnp.int32)]
    ).reshape(NPAD, 1)

    degh = _sc_deg(dstp)
    deg0 = degh[0, :, 0:1]
    deg1 = degh[1, :, 0:1]

    g1, dis = _tc1(xp, W1, deg0, deg1)
    p = _sc_agg(g1, srcp, dstp)
    g2 = _tc_mid(p[0], p[1], g1, dis, W2, b1.reshape(1, H))
    p = _sc_agg(g2, srcp, dstp)
    g3 = _tc_mid(p[0], p[1], g2, dis, W3, b2.reshape(1, H))
    p = _sc_agg(g3, srcp, dstp)
    out = _tc_final(p[0], p[1], g3, dis, b3.reshape(1, H), batchp,
                    head_W, head_b.reshape(1, T))
    return out


# speedup vs baseline: 18.3191x; 1.4496x over previous
import functools

import jax
import jax.numpy as jnp
from jax import lax
from jax.experimental import pallas as pl
from jax.experimental.pallas import tpu as pltpu
from jax.experimental.pallas import tpu_sc as plsc

N = 10000
E = 320000
H = 128
G = 64
T = 8

NC = 2
NS = 16
NW = NC * NS
NPAD = 10240
EPAD = 327680
EPW = EPAD // NW
CH = 128
NCHUNK = EPW // CH
RPT = NPAD // NS
DCH = 128

_mesh = plsc.VectorSubcoreMesh(
    core_axis_name="c", subcore_axis_name="s", num_cores=NC, num_subcores=NS)


@functools.partial(
    pl.kernel,
    out_type=jax.ShapeDtypeStruct((NC, NPAD, H), jnp.float32),
    mesh=_mesh,
    scratch_types=[
        pltpu.VMEM((CH,), jnp.int32),
        pltpu.VMEM((CH, H), jnp.float32),
        pltpu.VMEM_SHARED((NPAD, H), jnp.float32),
    ],
)
def _sc_deg(dst_hbm, deg_hbm, didx, ones_v, acc):
    c = lax.axis_index("c")
    s = lax.axis_index("s")
    wid = c * NS + s

    def fill_zero(i, _):
        ones_v[i // 8, pl.ds((i % 8) * 16, 16)] = jnp.zeros((16,), jnp.float32)
        return 0
    lax.fori_loop(0, CH * (H // 16), fill_zero, 0)

    def zero_acc(j, _):
        pltpu.sync_copy(ones_v, acc.at[pl.ds(s * RPT + j * DCH, DCH)])
        return 0
    lax.fori_loop(0, RPT // DCH, zero_acc, 0)

    def fill_ones(i, _):
        ones_v[i // 8, pl.ds((i % 8) * 16, 16)] = jnp.ones((16,), jnp.float32)
        return 0
    lax.fori_loop(0, CH * (H // 16), fill_ones, 0)
    plsc.subcore_barrier()

    def body(i, _):
        base = wid * EPW + i * CH
        pltpu.sync_copy(dst_hbm.at[pl.ds(base, CH)], didx)
        pltpu.sync_copy(ones_v, acc.at[didx], add=True)
        return 0
    lax.fori_loop(0, NCHUNK, body, 0)

    plsc.subcore_barrier()

    def dump(j, _):
        r0 = s * RPT + j * DCH
        pltpu.sync_copy(acc.at[pl.ds(r0, DCH)], ones_v)
        pltpu.sync_copy(ones_v, deg_hbm.at[c, pl.ds(r0, DCH)])
        return 0
    lax.fori_loop(0, RPT // DCH, dump, 0)


@functools.partial(
    pl.kernel,
    out_type=jax.ShapeDtypeStruct((NC, NPAD, H), jnp.float32),
    mesh=_mesh,
    scratch_types=[
        pltpu.VMEM((CH,), jnp.int32),
        pltpu.VMEM((CH,), jnp.int32),
        pltpu.VMEM((CH,), jnp.int32),
        pltpu.VMEM((CH,), jnp.int32),
        pltpu.VMEM((CH, H), jnp.float32),
        pltpu.VMEM((CH, H), jnp.float32),
        pltpu.SemaphoreType.DMA,
        pltpu.SemaphoreType.DMA,
        pltpu.VMEM_SHARED((NPAD, H), jnp.float32),
    ],
)
def _sc_agg(g_hbm, src_hbm, dst_hbm, p_hbm, sidx0, sidx1, didx0, didx1,
            rows0, rows1, sem0, sem1, acc):
    c = lax.axis_index("c")
    s = lax.axis_index("s")
    wid = c * NS + s
    base0 = wid * EPW

    def fill_zero(i, _):
        rows0[i // 8, pl.ds((i % 8) * 16, 16)] = jnp.zeros((16,), jnp.float32)
        return 0
    lax.fori_loop(0, CH * (H // 16), fill_zero, 0)

    def zero_acc(j, _):
        pltpu.sync_copy(rows0, acc.at[pl.ds(s * RPT + j * DCH, DCH)])
        return 0
    lax.fori_loop(0, RPT // DCH, zero_acc, 0)
    plsc.subcore_barrier()

    pltpu.sync_copy(src_hbm.at[pl.ds(base0, CH)], sidx0)
    pltpu.async_copy(g_hbm.at[sidx0], rows0, sem0)

    def body(j, _):
        i0 = 2 * j
        pltpu.sync_copy(src_hbm.at[pl.ds(base0 + (i0 + 1) * CH, CH)], sidx1)
        cp1 = pltpu.async_copy(g_hbm.at[sidx1], rows1, sem1)
        pltpu.make_async_copy(g_hbm.at[sidx0], rows0, sem0).wait()
        pltpu.sync_copy(dst_hbm.at[pl.ds(base0 + i0 * CH, CH)], didx0)
        pltpu.sync_copy(rows0, acc.at[didx0], add=True)
        nxt = jnp.minimum(i0 + 2, NCHUNK - 1)
        pltpu.sync_copy(src_hbm.at[pl.ds(base0 + nxt * CH, CH)], sidx0)
        pltpu.async_copy(g_hbm.at[sidx0], rows0, sem0)
        cp1.wait()
        pltpu.sync_copy(dst_hbm.at[pl.ds(base0 + (i0 + 1) * CH, CH)], didx1)
        pltpu.sync_copy(rows1, acc.at[didx1], add=True)
        return 0
    lax.fori_loop(0, NCHUNK // 2, body, 0)

    pltpu.make_async_copy(g_hbm.at[sidx0], rows0, sem0).wait()
    plsc.subcore_barrier()

    def dump(j, _):
        r0 = s * RPT + j * DCH
        pltpu.sync_copy(acc.at[pl.ds(r0, DCH)], rows0)
        pltpu.sync_copy(rows0, p_hbm.at[c, pl.ds(r0, DCH)])
        return 0
    lax.fori_loop(0, RPT // DCH, dump, 0)


BR = 1024
GRID = NPAD // BR


def _tc1_body(x_ref, w_ref, d0_ref, d1_ref, g_ref, dis_ref):
    deg = d0_ref[...] + d1_ref[...] + 1.0
    dis = lax.rsqrt(deg)
    t = jnp.dot(x_ref[...], w_ref[...], preferred_element_type=jnp.float32)
    g_ref[...] = t * dis
    dis_ref[...] = dis


_tc1 = pl.pallas_call(
    _tc1_body,
    grid=(GRID,),
    in_specs=[
        pl.BlockSpec((BR, H), lambda i: (i, 0)),
        pl.BlockSpec((H, H), lambda i: (0, 0)),
        pl.BlockSpec((BR, 1), lambda i: (i, 0)),
        pl.BlockSpec((BR, 1), lambda i: (i, 0)),
    ],
    out_specs=[
        pl.BlockSpec((BR, H), lambda i: (i, 0)),
        pl.BlockSpec((BR, 1), lambda i: (i, 0)),
    ],
    out_shape=[
        jax.ShapeDtypeStruct((NPAD, H), jnp.float32),
        jax.ShapeDtypeStruct((NPAD, 1), jnp.float32),
    ],
)


def _tc_mid_body(p0_ref, p1_ref, g_ref, dis_ref, w_ref, b_ref, out_ref):
    d = dis_ref[...]
    h = jnp.maximum(d * (p0_ref[...] + p1_ref[...] + g_ref[...]) + b_ref[...], 0.0)
    out_ref[...] = d * jnp.dot(h, w_ref[...], preferred_element_type=jnp.float32)


_tc_mid = pl.pallas_call(
    _tc_mid_body,
    grid=(GRID,),
    in_specs=[
        pl.BlockSpec((BR, H), lambda i: (i, 0)),
        pl.BlockSpec((BR, H), lambda i: (i, 0)),
        pl.BlockSpec((BR, H), lambda i: (i, 0)),
        pl.BlockSpec((BR, 1), lambda i: (i, 0)),
        pl.BlockSpec((H, H), lambda i: (0, 0)),
        pl.BlockSpec((1, H), lambda i: (0, 0)),
    ],
    out_specs=pl.BlockSpec((BR, H), lambda i: (i, 0)),
    out_shape=jax.ShapeDtypeStruct((NPAD, H), jnp.float32),
)


def _tc_final_body(p0_ref, p1_ref, g_ref, dis_ref, b_ref, batch_ref, hw_ref,
                   hb_ref, out_ref, sums, cnts):
    i = pl.program_id(0)

    @pl.when(i == 0)
    def _():
        sums[...] = jnp.zeros_like(sums)
        cnts[...] = jnp.zeros_like(cnts)

    d = dis_ref[...]
    h3 = d * (p0_ref[...] + p1_ref[...] + g_ref[...]) + b_ref[...]
    onehot = (batch_ref[...] == lax.broadcasted_iota(jnp.int32, (BR, G), 1)
              ).astype(jnp.float32)
    sums[...] += lax.dot_general(onehot, h3, (((0,), (0,)), ((), ())),
                                 preferred_element_type=jnp.float32)
    cnts[...] += lax.dot_general(onehot, jnp.ones((BR, H), jnp.float32),
                                 (((0,), (0,)), ((), ())),
                                 preferred_element_type=jnp.float32)

    @pl.when(i == GRID - 1)
    def _():
        pooled = sums[...] / jnp.maximum(cnts[...], 1.0)
        out_ref[...] = jnp.dot(pooled, hw_ref[...],
                               preferred_element_type=jnp.float32) + hb_ref[...]


_tc_final = pl.pallas_call(
    _tc_final_body,
    grid=(GRID,),
    in_specs=[
        pl.BlockSpec((BR, H), lambda i: (i, 0)),
        pl.BlockSpec((BR, H), lambda i: (i, 0)),
        pl.BlockSpec((BR, H), lambda i: (i, 0)),
        pl.BlockSpec((BR, 1), lambda i: (i, 0)),
        pl.BlockSpec((1, H), lambda i: (0, 0)),
        pl.BlockSpec((BR, 1), lambda i: (i, 0)),
        pl.BlockSpec((H, T), lambda i: (0, 0)),
        pl.BlockSpec((1, T), lambda i: (0, 0)),
    ],
    out_specs=pl.BlockSpec((G, T), lambda i: (0, 0)),
    out_shape=jax.ShapeDtypeStruct((G, T), jnp.float32),
    scratch_shapes=[
        pltpu.VMEM((G, H), jnp.float32),
        pltpu.VMEM((G, H), jnp.float32),
    ],
)


def kernel(x, edge_index, batch, r_target, W1, b1, W2, b2, W3, b3, head_W, head_b):
    del r_target
    pad_idx = N + jnp.arange(EPAD - E, dtype=jnp.int32) % (NPAD - N)
    srcp = jnp.concatenate([edge_index[0].astype(jnp.int32), pad_idx])
    dstp = jnp.concatenate([edge_index[1].astype(jnp.int32), pad_idx])
    xp = jnp.zeros((NPAD, H), jnp.float32).at[:N].set(x)
    batchp = jnp.concatenate(
        [batch.astype(jnp.int32), jnp.full((NPAD - N,), G, jnp.int32)]
    ).reshape(NPAD, 1)

    degh = _sc_deg(dstp)
    deg0 = degh[0, :, 0:1]
    deg1 = degh[1, :, 0:1]

    g1, dis = _tc1(xp, W1, deg0, deg1)
    p = _sc_agg(g1, srcp, dstp)
    g2 = _tc_mid(p[0], p[1], g1, dis, W2, b1.reshape(1, H))
    p = _sc_agg(g2, srcp, dstp)
    g3 = _tc_mid(p[0], p[1], g2, dis, W3, b2.reshape(1, H))
    p = _sc_agg(g3, srcp, dstp)
    out = _tc_final(p[0], p[1], g3, dis, b3.reshape(1, H), batchp,
                    head_W, head_b.reshape(1, T))
    return out


# speedup vs baseline: 21.7757x; 1.1887x over previous
import functools

import jax
import jax.numpy as jnp
from jax import lax
from jax.experimental import pallas as pl
from jax.experimental.pallas import tpu as pltpu
from jax.experimental.pallas import tpu_sc as plsc

N = 10000
E = 320000
H = 128
G = 64
T = 8

NC = 2
NS = 16
NW = NC * NS
NPAD = 10240
EPAD = 327680
EPW = EPAD // NW
CH = 128
NCHUNK = EPW // CH
RPT = NPAD // NS
DCH = 128

_mesh = plsc.VectorSubcoreMesh(
    core_axis_name="c", subcore_axis_name="s", num_cores=NC, num_subcores=NS)


@functools.partial(
    pl.kernel,
    out_type=jax.ShapeDtypeStruct((NC, NPAD, H), jnp.float32),
    mesh=_mesh,
    scratch_types=[
        pltpu.VMEM((NCHUNK, CH), jnp.int32),
        pltpu.VMEM((CH, H), jnp.float32),
        pltpu.VMEM_SHARED((NPAD, H), jnp.float32),
    ],
)
def _sc_deg(dst_hbm, deg_hbm, didx, ones_v, acc):
    c = lax.axis_index("c")
    s = lax.axis_index("s")
    wid = c * NS + s

    def fill_zero(i, _):
        ones_v[i // 8, pl.ds((i % 8) * 16, 16)] = jnp.zeros((16,), jnp.float32)
        return 0
    lax.fori_loop(0, CH * (H // 16), fill_zero, 0)

    def zero_acc(j, _):
        pltpu.sync_copy(ones_v, acc.at[pl.ds(s * RPT + j * DCH, DCH)])
        return 0
    lax.fori_loop(0, RPT // DCH, zero_acc, 0)

    def fill_ones(i, _):
        ones_v[i // 8, pl.ds((i % 8) * 16, 16)] = jnp.ones((16,), jnp.float32)
        return 0
    lax.fori_loop(0, CH * (H // 16), fill_ones, 0)

    pltpu.sync_copy(dst_hbm.at[pl.ds(wid * NCHUNK, NCHUNK)], didx)
    plsc.subcore_barrier()

    def body(i, _):
        pltpu.sync_copy(ones_v, acc.at[didx.at[i]], add=True)
        return 0
    lax.fori_loop(0, NCHUNK, body, 0)

    plsc.subcore_barrier()

    def dump(j, _):
        r0 = s * RPT + j * DCH
        pltpu.sync_copy(acc.at[pl.ds(r0, DCH)], ones_v)
        pltpu.sync_copy(ones_v, deg_hbm.at[c, pl.ds(r0, DCH)])
        return 0
    lax.fori_loop(0, RPT // DCH, dump, 0)


@functools.partial(
    pl.kernel,
    out_type=jax.ShapeDtypeStruct((NC, NPAD, H), jnp.float32),
    mesh=_mesh,
    scratch_types=[
        pltpu.VMEM((CH,), jnp.int32),
        pltpu.VMEM((CH,), jnp.int32),
        pltpu.VMEM((NCHUNK, CH), jnp.int32),
        pltpu.VMEM((CH, H), jnp.float32),
        pltpu.VMEM((CH, H), jnp.float32),
        pltpu.SemaphoreType.DMA,
        pltpu.SemaphoreType.DMA,
        pltpu.VMEM_SHARED((NPAD, H), jnp.float32),
    ],
)
def _sc_agg(g_hbm, src_hbm, dst_hbm, p_hbm, sidx0, sidx1, didx,
            rows0, rows1, sem0, sem1, acc):
    c = lax.axis_index("c")
    s = lax.axis_index("s")
    wid = c * NS + s

    def fill_zero(i, _):
        rows0[i // 8, pl.ds((i % 8) * 16, 16)] = jnp.zeros((16,), jnp.float32)
        return 0
    lax.fori_loop(0, CH * (H // 16), fill_zero, 0)

    def zero_acc(j, _):
        pltpu.sync_copy(rows0, acc.at[pl.ds(s * RPT + j * DCH, DCH)])
        return 0
    lax.fori_loop(0, RPT // DCH, zero_acc, 0)

    pltpu.sync_copy(dst_hbm.at[pl.ds(wid * NCHUNK, NCHUNK)], didx)
    plsc.subcore_barrier()

    pltpu.sync_copy(src_hbm.at[wid * NCHUNK], sidx0)
    pltpu.async_copy(g_hbm.at[sidx0], rows0, sem0)

    def body(j, _):
        i0 = 2 * j
        pltpu.sync_copy(src_hbm.at[wid * NCHUNK + i0 + 1], sidx1)
        cp1 = pltpu.async_copy(g_hbm.at[sidx1], rows1, sem1)
        pltpu.make_async_copy(g_hbm.at[sidx0], rows0, sem0).wait()
        pltpu.sync_copy(rows0, acc.at[didx.at[i0]], add=True)
        nxt = jnp.minimum(i0 + 2, NCHUNK - 1)
        pltpu.sync_copy(src_hbm.at[wid * NCHUNK + nxt], sidx0)
        pltpu.async_copy(g_hbm.at[sidx0], rows0, sem0)
        cp1.wait()
        pltpu.sync_copy(rows1, acc.at[didx.at[i0 + 1]], add=True)
        return 0
    lax.fori_loop(0, NCHUNK // 2, body, 0)

    pltpu.make_async_copy(g_hbm.at[sidx0], rows0, sem0).wait()
    plsc.subcore_barrier()

    def dump(j, _):
        r0 = s * RPT + j * DCH
        pltpu.sync_copy(acc.at[pl.ds(r0, DCH)], rows0)
        pltpu.sync_copy(rows0, p_hbm.at[c, pl.ds(r0, DCH)])
        return 0
    lax.fori_loop(0, RPT // DCH, dump, 0)


BR = 1024
GRID = NPAD // BR


def _tc1_body(x_ref, w_ref, d0_ref, d1_ref, g_ref, dis_ref):
    deg = d0_ref[...] + d1_ref[...] + 1.0
    dis = lax.rsqrt(deg)
    t = jnp.dot(x_ref[...], w_ref[...], preferred_element_type=jnp.float32)
    g_ref[...] = t * dis
    dis_ref[...] = dis


_tc1 = pl.pallas_call(
    _tc1_body,
    grid=(GRID,),
    in_specs=[
        pl.BlockSpec((BR, H), lambda i: (i, 0)),
        pl.BlockSpec((H, H), lambda i: (0, 0)),
        pl.BlockSpec((BR, 1), lambda i: (i, 0)),
        pl.BlockSpec((BR, 1), lambda i: (i, 0)),
    ],
    out_specs=[
        pl.BlockSpec((BR, H), lambda i: (i, 0)),
        pl.BlockSpec((BR, 1), lambda i: (i, 0)),
    ],
    out_shape=[
        jax.ShapeDtypeStruct((NPAD, H), jnp.float32),
        jax.ShapeDtypeStruct((NPAD, 1), jnp.float32),
    ],
)


def _tc_mid_body(p0_ref, p1_ref, g_ref, dis_ref, w_ref, b_ref, out_ref):
    d = dis_ref[...]
    h = jnp.maximum(d * (p0_ref[...] + p1_ref[...] + g_ref[...]) + b_ref[...], 0.0)
    out_ref[...] = d * jnp.dot(h, w_ref[...], preferred_element_type=jnp.float32)


_tc_mid = pl.pallas_call(
    _tc_mid_body,
    grid=(GRID,),
    in_specs=[
        pl.BlockSpec((BR, H), lambda i: (i, 0)),
        pl.BlockSpec((BR, H), lambda i: (i, 0)),
        pl.BlockSpec((BR, H), lambda i: (i, 0)),
        pl.BlockSpec((BR, 1), lambda i: (i, 0)),
        pl.BlockSpec((H, H), lambda i: (0, 0)),
        pl.BlockSpec((1, H), lambda i: (0, 0)),
    ],
    out_specs=pl.BlockSpec((BR, H), lambda i: (i, 0)),
    out_shape=jax.ShapeDtypeStruct((NPAD, H), jnp.float32),
)


def _tc_final_body(p0_ref, p1_ref, g_ref, dis_ref, b_ref, batch_ref, hw_ref,
                   hb_ref, out_ref, sums, cnts):
    i = pl.program_id(0)

    @pl.when(i == 0)
    def _():
        sums[...] = jnp.zeros_like(sums)
        cnts[...] = jnp.zeros_like(cnts)

    d = dis_ref[...]
    h3 = d * (p0_ref[...] + p1_ref[...] + g_ref[...]) + b_ref[...]
    onehot = (batch_ref[...] == lax.broadcasted_iota(jnp.int32, (BR, G), 1)
              ).astype(jnp.float32)
    sums[...] += lax.dot_general(onehot, h3, (((0,), (0,)), ((), ())),
                                 preferred_element_type=jnp.float32)
    cnts[...] += lax.dot_general(onehot, jnp.ones((BR, H), jnp.float32),
                                 (((0,), (0,)), ((), ())),
                                 preferred_element_type=jnp.float32)

    @pl.when(i == GRID - 1)
    def _():
        pooled = sums[...] / jnp.maximum(cnts[...], 1.0)
        out_ref[...] = jnp.dot(pooled, hw_ref[...],
                               preferred_element_type=jnp.float32) + hb_ref[...]


_tc_final = pl.pallas_call(
    _tc_final_body,
    grid=(GRID,),
    in_specs=[
        pl.BlockSpec((BR, H), lambda i: (i, 0)),
        pl.BlockSpec((BR, H), lambda i: (i, 0)),
        pl.BlockSpec((BR, H), lambda i: (i, 0)),
        pl.BlockSpec((BR, 1), lambda i: (i, 0)),
        pl.BlockSpec((1, H), lambda i: (0, 0)),
        pl.BlockSpec((BR, 1), lambda i: (i, 0)),
        pl.BlockSpec((H, T), lambda i: (0, 0)),
        pl.BlockSpec((1, T), lambda i: (0, 0)),
    ],
    out_specs=pl.BlockSpec((G, T), lambda i: (0, 0)),
    out_shape=jax.ShapeDtypeStruct((G, T), jnp.float32),
    scratch_shapes=[
        pltpu.VMEM((G, H), jnp.float32),
        pltpu.VMEM((G, H), jnp.float32),
    ],
)


def kernel(x, edge_index, batch, r_target, W1, b1, W2, b2, W3, b3, head_W, head_b):
    del r_target
    pad_idx = N + jnp.arange(EPAD - E, dtype=jnp.int32) % (NPAD - N)
    srcp = jnp.concatenate([edge_index[0].astype(jnp.int32), pad_idx]
                           ).reshape(EPAD // CH, CH)
    dstp = jnp.concatenate([edge_index[1].astype(jnp.int32), pad_idx]
                           ).reshape(EPAD // CH, CH)
    xp = jnp.zeros((NPAD, H), jnp.float32).at[:N].set(x)
    batchp = jnp.concatenate(
        [batch.astype(jnp.int32), jnp.full((NPAD - N,), G, jnp.int32)]
    ).reshape(NPAD, 1)

    degh = _sc_deg(dstp)
    deg0 = degh[0, :, 0:1]
    deg1 = degh[1, :, 0:1]

    g1, dis = _tc1(xp, W1, deg0, deg1)
    p = _sc_agg(g1, srcp, dstp)
    g2 = _tc_mid(p[0], p[1], g1, dis, W2, b1.reshape(1, H))
    p = _sc_agg(g2, srcp, dstp)
    g3 = _tc_mid(p[0], p[1], g2, dis, W3, b2.reshape(1, H))
    p = _sc_agg(g3, srcp, dstp)
    out = _tc_final(p[0], p[1], g3, dis, b3.reshape(1, H), batchp,
                    head_W, head_b.reshape(1, T))
    return out


# speedup vs baseline: 23.7949x; 1.0927x over previous
import functools

import jax
import jax.numpy as jnp
from jax import lax
from jax.experimental import pallas as pl
from jax.experimental.pallas import tpu as pltpu
from jax.experimental.pallas import tpu_sc as plsc

N = 10000
E = 320000
H = 128
G = 64
T = 8

NC = 2
NS = 16
NW = NC * NS
NPAD = 10240
EPAD = 327680
EPW = EPAD // NW
CH = 128
NCHUNK = EPW // CH
RPT = NPAD // NS
DCH = 128

_mesh = plsc.VectorSubcoreMesh(
    core_axis_name="c", subcore_axis_name="s", num_cores=NC, num_subcores=NS)


@functools.partial(
    pl.kernel,
    out_type=jax.ShapeDtypeStruct((NC, NPAD, H), jnp.float32),
    mesh=_mesh,
    scratch_types=[
        pltpu.VMEM((NCHUNK, CH), jnp.int32),
        pltpu.VMEM((CH, H), jnp.float32),
        pltpu.SemaphoreType.DMA,
        pltpu.SemaphoreType.DMA,
        pltpu.VMEM_SHARED((NPAD, H), jnp.float32),
    ],
)
def _sc_deg(dst_hbm, deg_hbm, didx, ones_v, sd0, sd1, acc):
    c = lax.axis_index("c")
    s = lax.axis_index("s")
    wid = c * NS + s

    def fill_zero(i, _):
        ones_v[i // 8, pl.ds((i % 8) * 16, 16)] = jnp.zeros((16,), jnp.float32)
        return 0
    lax.fori_loop(0, CH * (H // 16), fill_zero, 0)

    def zero_acc(j, _):
        pltpu.sync_copy(ones_v, acc.at[pl.ds(s * RPT + j * DCH, DCH)])
        return 0
    lax.fori_loop(0, RPT // DCH, zero_acc, 0)

    def fill_ones(i, _):
        ones_v[i // 8, pl.ds((i % 8) * 16, 16)] = jnp.ones((16,), jnp.float32)
        return 0
    lax.fori_loop(0, CH * (H // 16), fill_ones, 0)

    pltpu.sync_copy(dst_hbm.at[pl.ds(wid * NCHUNK, NCHUNK)], didx)
    plsc.subcore_barrier()

    sd = [sd0, sd1]

    def body(j, _):
        for k in range(2):
            ck = 2 * j + k

            @pl.when(ck >= 2)
            def _():
                pltpu.make_async_copy(ones_v, acc.at[didx.at[ck - 2]],
                                      sd[k]).wait()
            pltpu.async_copy(ones_v, acc.at[didx.at[ck]], sd[k], add=True)
        return 0
    lax.fori_loop(0, NCHUNK // 2, body, 0)

    pltpu.make_async_copy(ones_v, acc.at[didx.at[NCHUNK - 2]], sd0).wait()
    pltpu.make_async_copy(ones_v, acc.at[didx.at[NCHUNK - 1]], sd1).wait()
    plsc.subcore_barrier()

    def dump(j, _):
        r0 = s * RPT + j * DCH
        pltpu.sync_copy(acc.at[pl.ds(r0, DCH)], ones_v)
        pltpu.sync_copy(ones_v, deg_hbm.at[c, pl.ds(r0, DCH)])
        return 0
    lax.fori_loop(0, RPT // DCH, dump, 0)


@functools.partial(
    pl.kernel,
    out_type=jax.ShapeDtypeStruct((NC, NPAD, H), jnp.float32),
    mesh=_mesh,
    scratch_types=[
        pltpu.VMEM((4, CH), jnp.int32),
        pltpu.VMEM((NCHUNK, CH), jnp.int32),
        pltpu.VMEM((CH, H), jnp.float32),
        pltpu.VMEM((CH, H), jnp.float32),
        pltpu.SemaphoreType.DMA,
        pltpu.SemaphoreType.DMA,
        pltpu.SemaphoreType.DMA,
        pltpu.SemaphoreType.DMA,
        pltpu.SemaphoreType.DMA,
        pltpu.SemaphoreType.DMA,
        pltpu.VMEM_SHARED((NPAD, H), jnp.float32),
    ],
)
def _sc_agg(g_hbm, src_hbm, dst_hbm, p_hbm, sidx, didx, rows0, rows1,
            sg0, sg1, si0, si1, si2, si3, acc):
    c = lax.axis_index("c")
    s = lax.axis_index("s")
    wid = c * NS + s
    sg = [sg0, sg1]
    si = [si0, si1, si2, si3]

    def fill_zero(i, _):
        rows0[i // 8, pl.ds((i % 8) * 16, 16)] = jnp.zeros((16,), jnp.float32)
        return 0
    lax.fori_loop(0, CH * (H // 16), fill_zero, 0)

    def zero_acc(j, _):
        pltpu.sync_copy(rows0, acc.at[pl.ds(s * RPT + j * DCH, DCH)])
        return 0
    lax.fori_loop(0, RPT // DCH, zero_acc, 0)

    pltpu.sync_copy(dst_hbm.at[pl.ds(wid * NCHUNK, NCHUNK)], didx)

    rows = [rows0, rows1]
    pltpu.sync_copy(src_hbm.at[wid * NCHUNK], sidx.at[0])
    pltpu.sync_copy(src_hbm.at[wid * NCHUNK + 1], sidx.at[1])
    pltpu.async_copy(g_hbm.at[sidx.at[0]], rows0, sg0)
    pltpu.async_copy(g_hbm.at[sidx.at[1]], rows1, sg1)
    plsc.subcore_barrier()

    def body(j, _):
        c0 = 4 * j
        for k in range(4):
            ck = c0 + k
            kp = k % 2
            kn = (k + 2) % 4
            @pl.when(ck + 2 < NCHUNK)
            def _():
                pltpu.async_copy(src_hbm.at[wid * NCHUNK + ck + 2],
                                 sidx.at[kn], si[kn])
            pltpu.make_async_copy(g_hbm.at[sidx.at[k]], rows[kp], sg[kp]).wait()
            pltpu.sync_copy(rows[kp], acc.at[didx.at[ck]], add=True)
            @pl.when(ck + 2 < NCHUNK)
            def _():
                pltpu.make_async_copy(src_hbm.at[wid * NCHUNK + ck + 2],
                                      sidx.at[kn], si[kn]).wait()
                pltpu.async_copy(g_hbm.at[sidx.at[kn]], rows[kp], sg[kp])
        return 0
    lax.fori_loop(0, NCHUNK // 4, body, 0)

    plsc.subcore_barrier()

    def dump(j, _):
        r0 = s * RPT + j * DCH
        pltpu.sync_copy(acc.at[pl.ds(r0, DCH)], rows0)
        pltpu.sync_copy(rows0, p_hbm.at[c, pl.ds(r0, DCH)])
        return 0
    lax.fori_loop(0, RPT // DCH, dump, 0)


BR = 1024
GRID = NPAD // BR


def _tc1_body(x_ref, w_ref, d0_ref, d1_ref, g_ref, dis_ref):
    deg = d0_ref[...] + d1_ref[...] + 1.0
    dis = lax.rsqrt(deg)
    t = jnp.dot(x_ref[...], w_ref[...], preferred_element_type=jnp.float32)
    g_ref[...] = t * dis
    dis_ref[...] = dis


_tc1 = pl.pallas_call(
    _tc1_body,
    grid=(GRID,),
    in_specs=[
        pl.BlockSpec((BR, H), lambda i: (i, 0)),
        pl.BlockSpec((H, H), lambda i: (0, 0)),
        pl.BlockSpec((BR, 1), lambda i: (i, 0)),
        pl.BlockSpec((BR, 1), lambda i: (i, 0)),
    ],
    out_specs=[
        pl.BlockSpec((BR, H), lambda i: (i, 0)),
        pl.BlockSpec((BR, 1), lambda i: (i, 0)),
    ],
    out_shape=[
        jax.ShapeDtypeStruct((NPAD, H), jnp.float32),
        jax.ShapeDtypeStruct((NPAD, 1), jnp.float32),
    ],
)


def _tc_mid_body(p0_ref, p1_ref, g_ref, dis_ref, w_ref, b_ref, out_ref):
    d = dis_ref[...]
    h = jnp.maximum(d * (p0_ref[...] + p1_ref[...] + g_ref[...]) + b_ref[...], 0.0)
    out_ref[...] = d * jnp.dot(h, w_ref[...], preferred_element_type=jnp.float32)


_tc_mid = pl.pallas_call(
    _tc_mid_body,
    grid=(GRID,),
    in_specs=[
        pl.BlockSpec((BR, H), lambda i: (i, 0)),
        pl.BlockSpec((BR, H), lambda i: (i, 0)),
        pl.BlockSpec((BR, H), lambda i: (i, 0)),
        pl.BlockSpec((BR, 1), lambda i: (i, 0)),
        pl.BlockSpec((H, H), lambda i: (0, 0)),
        pl.BlockSpec((1, H), lambda i: (0, 0)),
    ],
    out_specs=pl.BlockSpec((BR, H), lambda i: (i, 0)),
    out_shape=jax.ShapeDtypeStruct((NPAD, H), jnp.float32),
)


def _tc_final_body(p0_ref, p1_ref, g_ref, dis_ref, b_ref, batch_ref, hw_ref,
                   hb_ref, out_ref, sums, cnts):
    i = pl.program_id(0)

    @pl.when(i == 0)
    def _():
        sums[...] = jnp.zeros_like(sums)
        cnts[...] = jnp.zeros_like(cnts)

    d = dis_ref[...]
    h3 = d * (p0_ref[...] + p1_ref[...] + g_ref[...]) + b_ref[...]
    onehot = (batch_ref[...] == lax.broadcasted_iota(jnp.int32, (BR, G), 1)
              ).astype(jnp.float32)
    sums[...] += lax.dot_general(onehot, h3, (((0,), (0,)), ((), ())),
                                 preferred_element_type=jnp.float32)
    cnts[...] += lax.dot_general(onehot, jnp.ones((BR, H), jnp.float32),
                                 (((0,), (0,)), ((), ())),
                                 preferred_element_type=jnp.float32)

    @pl.when(i == GRID - 1)
    def _():
        pooled = sums[...] / jnp.maximum(cnts[...], 1.0)
        out_ref[...] = jnp.dot(pooled, hw_ref[...],
                               preferred_element_type=jnp.float32) + hb_ref[...]


_tc_final = pl.pallas_call(
    _tc_final_body,
    grid=(GRID,),
    in_specs=[
        pl.BlockSpec((BR, H), lambda i: (i, 0)),
        pl.BlockSpec((BR, H), lambda i: (i, 0)),
        pl.BlockSpec((BR, H), lambda i: (i, 0)),
        pl.BlockSpec((BR, 1), lambda i: (i, 0)),
        pl.BlockSpec((1, H), lambda i: (0, 0)),
        pl.BlockSpec((BR, 1), lambda i: (i, 0)),
        pl.BlockSpec((H, T), lambda i: (0, 0)),
        pl.BlockSpec((1, T), lambda i: (0, 0)),
    ],
    out_specs=pl.BlockSpec((G, T), lambda i: (0, 0)),
    out_shape=jax.ShapeDtypeStruct((G, T), jnp.float32),
    scratch_shapes=[
        pltpu.VMEM((G, H), jnp.float32),
        pltpu.VMEM((G, H), jnp.float32),
    ],
)


def kernel(x, edge_index, batch, r_target, W1, b1, W2, b2, W3, b3, head_W, head_b):
    del r_target
    pad_idx = N + jnp.arange(EPAD - E, dtype=jnp.int32) % (NPAD - N)
    srcp = jnp.concatenate([edge_index[0].astype(jnp.int32), pad_idx]
                           ).reshape(EPAD // CH, CH)
    dstp = jnp.concatenate([edge_index[1].astype(jnp.int32), pad_idx]
                           ).reshape(EPAD // CH, CH)
    xp = jnp.zeros((NPAD, H), jnp.float32).at[:N].set(x)
    batchp = jnp.concatenate(
        [batch.astype(jnp.int32), jnp.full((NPAD - N,), G, jnp.int32)]
    ).reshape(NPAD, 1)

    degh = _sc_deg(dstp)
    deg0 = degh[0, :, 0:1]
    deg1 = degh[1, :, 0:1]
    g1, dis = _tc1(xp, W1, deg0, deg1)
    p = _sc_agg(g1, srcp, dstp)
    g2 = _tc_mid(p[0], p[1], g1, dis, W2, b1.reshape(1, H))
    p = _sc_agg(g2, srcp, dstp)
    g3 = _tc_mid(p[0], p[1], g2, dis, W3, b2.reshape(1, H))
    p = _sc_agg(g3, srcp, dstp)
    out = _tc_final(p[0], p[1], g3, dis, b3.reshape(1, H), batchp,
                    head_W, head_b.reshape(1, T))
    return out
